# baseline (speedup 1.0000x reference)
import jax
import jax.numpy as jnp
from jax.experimental import pallas as pl

HIDDEN = 2048
N_EXP = 8
BLK_R = 4096
BLK_K = 512


def _tc_kernel(x_ref, w_ref, o_ref):
    j = pl.program_id(1)
    partial = jax.lax.dot_general(
        x_ref[...], w_ref[...],
        dimension_numbers=(((1,), (1,)), ((), ())),
        preferred_element_type=jnp.float32,
    )

    @pl.when(j == 0)
    def _():
        o_ref[...] = partial

    @pl.when(j > 0)
    def _():
        o_ref[...] += partial


def kernel(x, weight):
    xf = x.reshape(-1, HIDDEN)
    rows = xf.shape[0]
    out = pl.pallas_call(
        _tc_kernel,
        grid=(rows // BLK_R, HIDDEN // BLK_K),
        in_specs=[
            pl.BlockSpec((BLK_R, BLK_K), lambda i, j: (i, j)),
            pl.BlockSpec((N_EXP, BLK_K), lambda i, j: (0, j)),
        ],
        out_specs=pl.BlockSpec((BLK_R, N_EXP), lambda i, j: (i, 0)),
        out_shape=jax.ShapeDtypeStruct((rows, N_EXP), jnp.float32),
    )(xf, weight)
    return out
